# baseline (speedup 1.0000x reference)
import jax
import jax.numpy as jnp
from jax import lax
from jax.experimental import pallas as pl
from jax.experimental.pallas import tpu as pltpu
from jax.experimental.pallas import tpu_sc as plsc

MAXLEN = 200
EMB = 64
BATCH = 4096

NC = 2
NS = 16
NW = NC * NS

LB = MAXLEN // 8
BB = BATCH // 128
NBUF = 4
NPB = 2


def _emb_body(idx_hbm, tok_hbm, pos_hbm, out_hbm,
              pos_v, idx_all, r0, r1, r2, r3, p0, p1,
              g0, g1, g2, g3, o0, o1):
    rows = (r0, r1, r2, r3)
    planes = (p0, p1)
    gsem = (g0, g1, g2, g3)
    osem = (o0, o1)
    wid = lax.axis_index("s") * NC + lax.axis_index("c")

    pltpu.sync_copy(pos_hbm, pos_v)
    for lb in range(LB):
        pltpu.sync_copy(idx_hbm.at[lb, wid], idx_all.at[lb])

    def fire_gather(lb, ls, b):
        pltpu.async_copy(tok_hbm.at[idx_all.at[lb, ls]], rows[b], gsem[b])

    def drain_gather(b):
        pltpu.make_async_copy(tok_hbm.at[pl.ds(0, 128)], rows[b],
                              gsem[b]).wait()

    def drain_plane(q):
        for e1 in range(8):
            pltpu.make_async_copy(planes[q].at[pl.ds(8 * e1, 8), pl.ds(0, 128)],
                                  out_hbm.at[0, e1, wid], osem[q]).wait()

    fire_gather(0, 0, 0)
    fire_gather(0, 1, 1)

    iota16 = lax.iota(jnp.int32, 16)

    def lb_body(lb, carry):
        for ls in range(8):
            g_par = (lb << 3) + ls
            b = ls % NBUF
            q = ls % NPB
            l = g_par + 0

            ls2 = (ls + 2) % 8
            lb2 = lb + (1 if ls >= 6 else 0)
            if ls >= 6:
                @pl.when(lb2 < LB)
                def _pref():
                    fire_gather(lb2, ls2, (ls2 % NBUF))
            else:
                fire_gather(lb2, ls2, (ls2 % NBUF))

            drain_gather(b)

            if ls >= NPB:
                drain_plane(q)
            else:
                @pl.when(lb > 0)
                def _recyc():
                    drain_plane(q)

            row_r = rows[b]
            plane = planes[q]

            for c in range(EMB // 16):
                pos_vec = pos_v[l, pl.ds(16 * c, 16)]
                ev = iota16 + (16 * c)

                @plsc.parallel_loop(0, 128, 1, unroll=16)
                def b2_body(b2, _c=c, _ev=ev,
                            _pos=pos_vec, _row=row_r, _plane=plane):
                    bv = jnp.broadcast_to(b2, (16,))
                    vals = _row[b2, pl.ds(16 * _c, 16)]
                    plsc.store_scatter(_plane, [_ev, bv], vals + _pos)

            for e1 in range(8):
                pltpu.async_copy(plane.at[pl.ds(8 * e1, 8), pl.ds(0, 128)],
                                 out_hbm.at[l, e1, wid], osem[q])
        return carry

    lax.fori_loop(0, LB, lb_body, 0)
    for q in range(NPB):
        drain_plane(q)


@jax.jit
def kernel(x, token_table, pos_table):
    idx4 = (x.astype(jnp.int32).T
            .reshape(LB, 8, BB, 128).transpose(0, 2, 1, 3))
    mesh = plsc.VectorSubcoreMesh(core_axis_name="c", subcore_axis_name="s")
    out5 = pl.kernel(
        _emb_body,
        mesh=mesh,
        compiler_params=pltpu.CompilerParams(use_tc_tiling_on_sc=False,
                                             needs_layout_passes=False,
                                             disable_bounds_checks=True),
        out_type=jax.ShapeDtypeStruct((MAXLEN, 8, BB, 8, 128), jnp.float32),
        scratch_types=(
            [pltpu.VMEM((MAXLEN, EMB), jnp.float32),
             pltpu.VMEM((LB, 8, 128), jnp.int32)]
            + [pltpu.VMEM((128, EMB), jnp.float32)] * NBUF
            + [pltpu.VMEM((EMB, 129), jnp.float32)] * NPB
            + [pltpu.SemaphoreType.DMA] * (NBUF + NPB)
        ),
    )(idx4, token_table, pos_table)
    return out5.transpose(2, 4, 0, 1, 3).reshape(BATCH, MAXLEN, EMB)


# speedup vs baseline: 1.0858x; 1.0858x over previous
import jax
import jax.numpy as jnp
from jax import lax
from jax.experimental import pallas as pl
from jax.experimental.pallas import tpu as pltpu
from jax.experimental.pallas import tpu_sc as plsc

MAXLEN = 200
EMB = 64
BATCH = 4096

NC = 2
NS = 16
NW = NC * NS

LB = MAXLEN // 8
BB = BATCH // 128
NBUF = 4
NPB = 2


def _emb_body(idx_hbm, tok_hbm, pos_hbm, out_hbm,
              pos_v, idx_all, r0, r1, r2, r3, p0, p1,
              g0, g1, g2, g3, o0, o1):
    rows = (r0, r1, r2, r3)
    planes = (p0, p1)
    gsem = (g0, g1, g2, g3)
    osem = (o0, o1)
    wid = lax.axis_index("s") * NC + lax.axis_index("c")

    pltpu.sync_copy(pos_hbm, pos_v)
    for lb in range(LB):
        pltpu.sync_copy(idx_hbm.at[lb, wid], idx_all.at[lb])

    def fire_gather(lb, ls, b):
        pltpu.async_copy(tok_hbm.at[idx_all.at[lb, ls]], rows[b], gsem[b])

    def drain_gather(b):
        pltpu.make_async_copy(tok_hbm.at[pl.ds(0, 128)], rows[b],
                              gsem[b]).wait()

    def drain_plane(q):
        pltpu.make_async_copy(planes[q].at[:, :, pl.ds(0, 128)],
                              out_hbm.at[0, :, wid], osem[q]).wait()

    fire_gather(0, 0, 0)
    fire_gather(0, 1, 1)

    iota16 = lax.iota(jnp.int32, 16)

    def lb_body(lb, carry):
        for ls in range(8):
            g_par = (lb << 3) + ls
            b = ls % NBUF
            q = ls % NPB
            l = g_par + 0

            ls2 = (ls + 2) % 8
            lb2 = lb + (1 if ls >= 6 else 0)
            if ls >= 6:
                @pl.when(lb2 < LB)
                def _pref():
                    fire_gather(lb2, ls2, (ls2 % NBUF))
            else:
                fire_gather(lb2, ls2, (ls2 % NBUF))

            drain_gather(b)

            if ls >= NPB:
                drain_plane(q)
            else:
                @pl.when(lb > 0)
                def _recyc():
                    drain_plane(q)

            row_r = rows[b]
            plane = planes[q]

            for c in range(EMB // 16):
                pos_vec = pos_v[l, pl.ds(16 * c, 16)]
                ep = iota16 + (16 * c)
                e1v = ep >> 3
                e2v = ep & 7

                @plsc.parallel_loop(0, 128, 1, unroll=8)
                def b2_body(b2, _c=c, _e1v=e1v, _e2v=e2v,
                            _pos=pos_vec, _row=row_r, _plane=plane):
                    bv = jnp.broadcast_to(b2, (16,))
                    vals = _row[b2, pl.ds(16 * _c, 16)]
                    plsc.store_scatter(_plane, [_e1v, _e2v, bv], vals + _pos)

            pltpu.async_copy(plane.at[:, :, pl.ds(0, 128)],
                             out_hbm.at[l, :, wid], osem[q])
        return carry

    lax.fori_loop(0, LB, lb_body, 0)
    for q in range(NPB):
        drain_plane(q)


@jax.jit
def kernel(x, token_table, pos_table):
    idx4 = (x.astype(jnp.int32).T
            .reshape(LB, 8, BB, 128).transpose(0, 2, 1, 3))
    mesh = plsc.VectorSubcoreMesh(core_axis_name="c", subcore_axis_name="s")
    out5 = pl.kernel(
        _emb_body,
        mesh=mesh,
        compiler_params=pltpu.CompilerParams(use_tc_tiling_on_sc=False,
                                             needs_layout_passes=False,
                                             disable_bounds_checks=True),
        out_type=jax.ShapeDtypeStruct((MAXLEN, 8, BB, 8, 128), jnp.float32),
        scratch_types=(
            [pltpu.VMEM((MAXLEN, EMB), jnp.float32),
             pltpu.VMEM((LB, 8, 128), jnp.int32)]
            + [pltpu.VMEM((128, EMB), jnp.float32)] * NBUF
            + [pltpu.VMEM((8, 8, 129), jnp.float32)] * NPB
            + [pltpu.SemaphoreType.DMA] * (NBUF + NPB)
        ),
    )(idx4, token_table, pos_table)
    return out5.transpose(2, 4, 0, 1, 3).reshape(BATCH, MAXLEN, EMB)


# speedup vs baseline: 1.0885x; 1.0024x over previous
import jax
import jax.numpy as jnp
from jax import lax
from jax.experimental import pallas as pl
from jax.experimental.pallas import tpu as pltpu
from jax.experimental.pallas import tpu_sc as plsc

MAXLEN = 200
EMB = 64
BATCH = 4096

NC = 2
NS = 16
NW = NC * NS

LB = MAXLEN // 8
BB = BATCH // 128
NBUF = 4
NPB = 4


def _emb_body(idx_hbm, tok_hbm, pos_hbm, out_hbm,
              pos_v, idx_all, r0, r1, r2, r3, p0, p1, p2, p3,
              g0, g1, g2, g3, o0, o1, o2, o3):
    rows = (r0, r1, r2, r3)
    planes = (p0, p1, p2, p3)
    gsem = (g0, g1, g2, g3)
    osem = (o0, o1, o2, o3)
    wid = lax.axis_index("s") * NC + lax.axis_index("c")

    pltpu.sync_copy(pos_hbm, pos_v)
    for lb in range(LB):
        pltpu.sync_copy(idx_hbm.at[lb, wid], idx_all.at[lb])

    def fire_gather(lb, ls, b):
        pltpu.async_copy(tok_hbm.at[idx_all.at[lb, ls]], rows[b], gsem[b])

    def drain_gather(b):
        pltpu.make_async_copy(tok_hbm.at[pl.ds(0, 128)], rows[b],
                              gsem[b]).wait()

    def drain_plane(q):
        pltpu.make_async_copy(planes[q].at[:, :, pl.ds(0, 128)],
                              out_hbm.at[0, :, wid], osem[q]).wait()

    fire_gather(0, 0, 0)
    fire_gather(0, 1, 1)

    iota16 = lax.iota(jnp.int32, 16)

    def lb_body(lb, carry):
        for ls in range(8):
            g_par = (lb << 3) + ls
            b = ls % NBUF
            q = ls % NPB
            l = g_par + 0

            ls2 = (ls + 2) % 8
            lb2 = lb + (1 if ls >= 6 else 0)
            if ls >= 6:
                @pl.when(lb2 < LB)
                def _pref():
                    fire_gather(lb2, ls2, (ls2 % NBUF))
            else:
                fire_gather(lb2, ls2, (ls2 % NBUF))

            drain_gather(b)

            if ls >= NPB:
                drain_plane(q)
            else:
                @pl.when(lb > 0)
                def _recyc():
                    drain_plane(q)

            row_r = rows[b]
            plane = planes[q]

            for c in range(EMB // 16):
                pos_vec = pos_v[l, pl.ds(16 * c, 16)]
                ep = iota16 + (16 * c)
                e1v = ep >> 3
                e2v = ep & 7

                @plsc.parallel_loop(0, 128, 1, unroll=8)
                def b2_body(b2, _c=c, _e1v=e1v, _e2v=e2v,
                            _pos=pos_vec, _row=row_r, _plane=plane):
                    bv = jnp.broadcast_to(b2, (16,))
                    vals = _row[b2, pl.ds(16 * _c, 16)]
                    plsc.store_scatter(_plane, [_e1v, _e2v, bv], vals + _pos)

            pltpu.async_copy(plane.at[:, :, pl.ds(0, 128)],
                             out_hbm.at[l, :, wid], osem[q])
        return carry

    lax.fori_loop(0, LB, lb_body, 0)
    for q in range(NPB):
        drain_plane(q)


@jax.jit
def kernel(x, token_table, pos_table):
    idx4 = (x.astype(jnp.int32).T
            .reshape(LB, 8, BB, 128).transpose(0, 2, 1, 3))
    mesh = plsc.VectorSubcoreMesh(core_axis_name="c", subcore_axis_name="s")
    out5 = pl.kernel(
        _emb_body,
        mesh=mesh,
        compiler_params=pltpu.CompilerParams(use_tc_tiling_on_sc=False,
                                             needs_layout_passes=False,
                                             disable_bounds_checks=True),
        out_type=jax.ShapeDtypeStruct((MAXLEN, 8, BB, 8, 128), jnp.float32),
        scratch_types=(
            [pltpu.VMEM((MAXLEN, EMB), jnp.float32),
             pltpu.VMEM((LB, 8, 128), jnp.int32)]
            + [pltpu.VMEM((128, EMB), jnp.float32)] * NBUF
            + [pltpu.VMEM((8, 8, 129), jnp.float32)] * NPB
            + [pltpu.SemaphoreType.DMA] * (NBUF + NPB)
        ),
    )(idx4, token_table, pos_table)
    return out5.transpose(2, 4, 0, 1, 3).reshape(BATCH, MAXLEN, EMB)
